# baseline (speedup 1.0000x reference)
import jax
import jax.numpy as jnp
from jax import lax
from jax.experimental import pallas as pl
from jax.experimental.pallas import tpu as pltpu

_MU = 0.25
_B = 4096
_BLK_E = 2048
_BLK_D = 2048
_NCODE = 256
_EDIM = 32
_F32 = jnp.float32


def _dott(a, b):
    return lax.dot_general(a, b, (((1,), (1,)), ((), ())),
                           preferred_element_type=_F32)


def _dot(a, b):
    return lax.dot_general(a, b, (((1,), (0,)), ((), ())),
                           preferred_element_type=_F32)


def _enc_kernel(x_ref,
                we0, we1, we2, we3, we4,
                be0, be1, be2, be3, be4,
                cb0, cb1, cb2, cb3,
                xq_ref, loss_ref, idx_ref):
    enc = [(we0, be0), (we1, be1), (we2, be2), (we3, be3), (we4, be4)]
    cbs = [cb0, cb1, cb2, cb3]

    h = x_ref[:]
    for i, (w, b) in enumerate(enc):
        h = _dott(h, w[:]) + b[:]
        if i != len(enc) - 1:
            h = jnp.maximum(h, 0.0)

    res = h
    xq = jnp.zeros_like(res)
    sq_total = jnp.float32(0.0)
    idx_cols = []
    iota = lax.broadcasted_iota(jnp.int32, (_BLK_E, _NCODE), 1)
    for cb_ref in cbs:
        cb = cb_ref[:]
        rowsq = jnp.sum(res * res, axis=1, keepdims=True)
        cbsq = jnp.sum(cb * cb, axis=1)[None, :]
        d = (rowsq + cbsq) - 2.0 * _dott(res, cb)
        m = jnp.min(d, axis=1, keepdims=True)
        idx = jnp.min(jnp.where(d == m, iota, _NCODE), axis=1, keepdims=True)
        onehot = (iota == idx).astype(_F32)
        xr = _dot(onehot, cb)
        diff = xr - res
        sq_total += jnp.sum(diff * diff)
        res = res - xr
        xq = xq + xr
        idx_cols.append(idx)
    idx_ref[:] = jnp.concatenate(idx_cols, axis=1)
    xq_ref[:] = xq

    @pl.when(pl.program_id(0) == 0)
    def _():
        loss_ref[:, :] = jnp.zeros((1, 1), _F32)
    scale = (1.0 + _MU) / (len(cbs) * _B * _EDIM)
    loss_ref[:, :] += (scale * sq_total).reshape(1, 1)


def _dec_kernel(xq_ref,
                wd0, wd1, wd2, wd3, wd4,
                bd0, bd1, bd2, bd3, bd4,
                out_ref):
    dec = [(wd0, bd0), (wd1, bd1), (wd2, bd2), (wd3, bd3), (wd4, bd4)]
    h = xq_ref[:]
    for i, (w, b) in enumerate(dec):
        h = _dott(h, w[:]) + b[:]
        if i != len(dec) - 1:
            h = jnp.maximum(h, 0.0)
    out_ref[:] = h


@jax.jit
def kernel(x, We0, We1, We2, We3, We4, be0, be1, be2, be3, be4,
           Wd0, Wd1, Wd2, Wd3, Wd4, bd0, bd1, bd2, bd3, bd4,
           cb0, cb1, cb2, cb3):
    rep = lambda i: (0, 0)
    full = lambda a: pl.BlockSpec(a.shape, rep)
    row = lambda b: pl.BlockSpec((1, b.shape[0]), rep)

    xq, loss, idx = pl.pallas_call(
        _enc_kernel,
        grid=(_B // _BLK_E,),
        in_specs=[pl.BlockSpec((_BLK_E, x.shape[1]), lambda i: (i, 0))]
                 + [full(w) for w in (We0, We1, We2, We3, We4)]
                 + [row(b) for b in (be0, be1, be2, be3, be4)]
                 + [full(c) for c in (cb0, cb1, cb2, cb3)],
        out_specs=[
            pl.BlockSpec((_BLK_E, _EDIM), lambda i: (i, 0)),
            pl.BlockSpec((1, 1), rep),
            pl.BlockSpec((_BLK_E, 4), lambda i: (i, 0)),
        ],
        out_shape=[
            jax.ShapeDtypeStruct((_B, _EDIM), _F32),
            jax.ShapeDtypeStruct((1, 1), _F32),
            jax.ShapeDtypeStruct((_B, 4), jnp.int32),
        ],
        compiler_params=pltpu.CompilerParams(
            dimension_semantics=("arbitrary",),
        ),
    )(x, We0, We1, We2, We3, We4,
      be0.reshape(1, -1), be1.reshape(1, -1), be2.reshape(1, -1),
      be3.reshape(1, -1), be4.reshape(1, -1),
      cb0, cb1, cb2, cb3)

    out = pl.pallas_call(
        _dec_kernel,
        grid=(_B // _BLK_D,),
        in_specs=[pl.BlockSpec((_BLK_D, _EDIM), lambda i: (i, 0))]
                 + [full(w) for w in (Wd0, Wd1, Wd2, Wd3, Wd4)]
                 + [row(b) for b in (bd0, bd1, bd2, bd3, bd4)],
        out_specs=pl.BlockSpec((_BLK_D, Wd4.shape[0]), lambda i: (i, 0)),
        out_shape=jax.ShapeDtypeStruct((_B, Wd4.shape[0]), _F32),
        compiler_params=pltpu.CompilerParams(
            dimension_semantics=("arbitrary",),
        ),
    )(xq,
      Wd0, Wd1, Wd2, Wd3, Wd4,
      bd0.reshape(1, -1), bd1.reshape(1, -1), bd2.reshape(1, -1),
      bd3.reshape(1, -1), bd4.reshape(1, -1))
    return out, loss[0, 0], idx


# speedup vs baseline: 1.1299x; 1.1299x over previous
import jax
import jax.numpy as jnp
from jax import lax
from jax.experimental import pallas as pl
from jax.experimental.pallas import tpu as pltpu

_MU = 0.25
_B = 4096
_BLK = 1024
_NCODE = 256
_EDIM = 32
_F32 = jnp.float32


def _dot(a, b):
    return lax.dot_general(a, b, (((1,), (0,)), ((), ())),
                           preferred_element_type=_F32)


def _dott(a, b):
    return lax.dot_general(a, b, (((1,), (1,)), ((), ())),
                           preferred_element_type=_F32)


def _fwd_kernel(x_ref,
                we0, we1, we2, we3, we4,
                be0, be1, be2, be3, be4,
                wd0, wd1, wd2, wd3, wd4,
                bd0, bd1, bd2, bd3, bd4,
                cb0, cb1, cb2, cb3,
                out_ref, loss_ref, idx_ref):
    enc = [(we0, be0), (we1, be1), (we2, be2), (we3, be3), (we4, be4)]
    dec = [(wd0, bd0), (wd1, bd1), (wd2, bd2), (wd3, bd3), (wd4, bd4)]
    cbs = [cb0, cb1, cb2, cb3]

    h = x_ref[:]
    for i, (w, b) in enumerate(enc):
        h = _dott(h, w[:]) + b[:]
        if i != len(enc) - 1:
            h = jnp.maximum(h, 0.0)

    res = h
    xq = jnp.zeros_like(res)
    sq_total = jnp.float32(0.0)
    idx_cols = []
    iota = lax.broadcasted_iota(jnp.int32, (_BLK, _NCODE), 1)
    for cb_ref in cbs:
        cb = cb_ref[:]
        rowsq = jnp.sum(res * res, axis=1, keepdims=True)
        cbsq = jnp.sum(cb * cb, axis=1)[None, :]
        d = (rowsq + cbsq) - 2.0 * _dott(res, cb)
        m = jnp.min(d, axis=1, keepdims=True)
        idx = jnp.min(jnp.where(d == m, iota, _NCODE), axis=1, keepdims=True)
        onehot = (iota == idx).astype(_F32)
        xr = _dot(onehot, cb)
        diff = xr - res
        sq_total += jnp.sum(diff * diff)
        res = res - xr
        xq = xq + xr
        idx_cols.append(idx)
    idx_ref[:] = jnp.concatenate(idx_cols, axis=1)

    h = xq
    for i, (w, b) in enumerate(dec):
        h = _dott(h, w[:]) + b[:]
        if i != len(dec) - 1:
            h = jnp.maximum(h, 0.0)
    out_ref[:] = h

    @pl.when(pl.program_id(0) == 0)
    def _():
        loss_ref[:, :] = jnp.zeros((1, 1), _F32)
    scale = (1.0 + _MU) / (len(cbs) * _B * _EDIM)
    loss_ref[:, :] += (scale * sq_total).reshape(1, 1)


@jax.jit
def kernel(x, We0, We1, We2, We3, We4, be0, be1, be2, be3, be4,
           Wd0, Wd1, Wd2, Wd3, Wd4, bd0, bd1, bd2, bd3, bd4,
           cb0, cb1, cb2, cb3):
    rep = lambda i: (0, 0)
    full = lambda a: pl.BlockSpec(a.shape, rep)
    row = lambda b: pl.BlockSpec((1, b.shape[0]), rep)

    out, loss, idx = pl.pallas_call(
        _fwd_kernel,
        grid=(_B // _BLK,),
        in_specs=[pl.BlockSpec((_BLK, x.shape[1]), lambda i: (i, 0))]
                 + [full(w) for w in (We0, We1, We2, We3, We4)]
                 + [row(b) for b in (be0, be1, be2, be3, be4)]
                 + [full(w) for w in (Wd0, Wd1, Wd2, Wd3, Wd4)]
                 + [row(b) for b in (bd0, bd1, bd2, bd3, bd4)]
                 + [full(c) for c in (cb0, cb1, cb2, cb3)],
        out_specs=[
            pl.BlockSpec((_BLK, Wd4.shape[0]), lambda i: (i, 0)),
            pl.BlockSpec((1, 1), rep),
            pl.BlockSpec((_BLK, 4), lambda i: (i, 0)),
        ],
        out_shape=[
            jax.ShapeDtypeStruct((_B, Wd4.shape[0]), _F32),
            jax.ShapeDtypeStruct((1, 1), _F32),
            jax.ShapeDtypeStruct((_B, 4), jnp.int32),
        ],
        compiler_params=pltpu.CompilerParams(
            dimension_semantics=("arbitrary",),
        ),
    )(x, We0, We1, We2, We3, We4,
      be0.reshape(1, -1), be1.reshape(1, -1), be2.reshape(1, -1),
      be3.reshape(1, -1), be4.reshape(1, -1),
      Wd0, Wd1, Wd2, Wd3, Wd4,
      bd0.reshape(1, -1), bd1.reshape(1, -1), bd2.reshape(1, -1),
      bd3.reshape(1, -1), bd4.reshape(1, -1),
      cb0, cb1, cb2, cb3)
    return out, loss[0, 0], idx
